# baseline (speedup 1.0000x reference)
import functools

import jax
import jax.numpy as jnp
from jax import lax
from jax.experimental import pallas as pl
from jax.experimental.pallas import tpu as pltpu
from jax.experimental.pallas import tpu_sc as plsc


def _make_sc_kernel(N, C, K, NC, NS, L):
    NW = NC * NS
    rows_per_w = N // NW
    G = K // L

    mesh = plsc.VectorSubcoreMesh(core_axis_name="c", subcore_axis_name="s")

    @functools.partial(
        pl.kernel,
        mesh=mesh,
        out_type=jax.ShapeDtypeStruct((N * K,), jnp.float32),
        compiler_params=pltpu.CompilerParams(needs_layout_passes=False),
        scratch_types=[
            pltpu.VMEM((rows_per_w * C,), jnp.float32),
            pltpu.VMEM((rows_per_w * K,), jnp.float32),
            pltpu.VMEM((K,), jnp.int32),
            pltpu.VMEM((K,), jnp.float32),
            pltpu.VMEM((K,), jnp.float32),
        ],
    )
    def sc_kernel(y_hbm, w_hbm, b_hbm, onn_hbm, out_hbm, in_v, out_v, onn_v, w_v, b_v):
        wid = lax.axis_index("s") * NC + lax.axis_index("c")
        pltpu.sync_copy(onn_hbm, onn_v)
        pltpu.sync_copy(w_hbm, w_v)
        pltpu.sync_copy(b_hbm, b_v)

        base = wid * rows_per_w
        pltpu.sync_copy(y_hbm.at[pl.ds(base * C, rows_per_w * C)], in_v)

        onn_g = [onn_v[pl.ds(g * L, L)] for g in range(G)]
        w_g = [w_v[pl.ds(g * L, L)] for g in range(G)]
        b_g = [b_v[pl.ds(g * L, L)] for g in range(G)]

        def body(r, carry):
            rb = r * C
            ob = r * K
            for g in range(G):
                idx = onn_g[g] + rb
                v = plsc.load_gather(in_v, [idx])
                out_v[pl.ds(ob + g * L, L)] = v * w_g[g] + b_g[g]
            return carry

        lax.fori_loop(0, rows_per_w, body, 0)

        pltpu.sync_copy(out_v, out_hbm.at[pl.ds(base * K, rows_per_w * K)])

    return sc_kernel


def kernel(Y_full, weights, bias, output_node_order):
    N, C = Y_full.shape
    K = output_node_order.shape[0]
    info = plsc.get_sparse_core_info()
    NC, NS, L = info.num_cores, info.num_subcores, info.num_lanes

    sc_kernel = _make_sc_kernel(N, C, K, NC, NS, L)
    out_flat = sc_kernel(
        Y_full.reshape(-1),
        weights,
        bias,
        output_node_order.astype(jnp.int32),
    )
    return out_flat.reshape(N, K)
